# baseline (speedup 1.0000x reference)
import functools

import numpy as np
import jax
import jax.numpy as jnp
from jax import lax
from jax.experimental import pallas as pl
from jax.experimental.pallas import tpu as pltpu
from jax.experimental.pallas import tpu_sc as plsc

INPUT_DIM = 3
NUM_LEVELS = 16
LEVEL_DIM = 2
BASE_RES = 16
LOG2_HASHMAP = 19
BATCH = 262144

_RES = [BASE_RES * 2 ** i for i in range(NUM_LEVELS)]
_PARAMS = []
_OFFSET = []
_off = 0
for _i in range(NUM_LEVELS):
    _p = min(2 ** LOG2_HASHMAP, _RES[_i] ** INPUT_DIM)
    _p = int(np.ceil(_p / 32) * 32)
    _PARAMS.append(_p)
    _OFFSET.append(_off)
    _off += _p
TOTAL_ROWS = _off

_P1 = int(np.uint32(2654435761).astype(np.int32))
_P2 = int(np.uint32(805459861).astype(np.int32))

NC, NS = 2, 16
NW = NC * NS
CHUNK = 512
PW = BATCH // NW
NCHUNKS = PW // CHUNK
NIDX = 8 * CHUNK
NGROUP = CHUNK // 16


def _sc_body(x_hbm, y_hbm, z_hbm, emb_hbm, out_hbm,
             xv, yv, zv, idx_a, idx_b, gat_a, gat_b, out_v, sem_a, sem_b):
    wid = lax.axis_index("s") * NC + lax.axis_index("c")
    iota = lax.iota(jnp.int32, 16)
    col0 = jnp.zeros((16,), jnp.int32)
    col1 = jnp.ones((16,), jnp.int32)
    idx_bufs = (idx_a, idx_b)
    gat_bufs = (gat_a, gat_b)
    sems = (sem_a, sem_b)

    def chunk_body(ci, carry):
        base = (wid * NCHUNKS + ci) * CHUNK
        pltpu.sync_copy(x_hbm.at[pl.ds(base, CHUNK)], xv)
        pltpu.sync_copy(y_hbm.at[pl.ds(base, CHUNK)], yv)
        pltpu.sync_copy(z_hbm.at[pl.ds(base, CHUNK)], zv)

        def gen_idx(level, idx_ref):
            res = float(_RES[level])
            mask = _PARAMS[level] - 1
            off = _OFFSET[level]

            def g_body(g, c):
                s = g * 16
                xi = (xv[pl.ds(s, 16)] * res).astype(jnp.int32)
                yi = (yv[pl.ds(s, 16)] * res).astype(jnp.int32)
                zi = (zv[pl.ds(s, 16)] * res).astype(jnp.int32)
                a0, b0 = xi, xi + 1
                a1 = yi * _P1
                b1 = a1 + _P1
                a2 = zi * _P2
                b2 = a2 + _P2
                e00 = a0 ^ a1
                e10 = b0 ^ a1
                e01 = a0 ^ b1
                e11 = b0 ^ b1
                pairs = (e00, e10, e01, e11)
                for corner in range(8):
                    h = pairs[corner & 3] ^ (b2 if corner & 4 else a2)
                    idx_ref[pl.ds(corner * CHUNK + s, 16)] = (h & mask) + off
                return c

            lax.fori_loop(jnp.int32(0), jnp.int32(NGROUP), g_body, 0)

        def fire(slot):
            return pltpu.async_copy(emb_hbm.at[idx_bufs[slot]],
                                    gat_bufs[slot], sems[slot])

        def accum(level, gat_ref):
            res = float(_RES[level])

            def g_body(g, c):
                s = g * 16
                x = xv[pl.ds(s, 16)] * res
                y = yv[pl.ds(s, 16)] * res
                z = zv[pl.ds(s, 16)] * res
                fx = x - x.astype(jnp.int32).astype(jnp.float32)
                fy = y - y.astype(jnp.int32).astype(jnp.float32)
                fz = z - z.astype(jnp.int32).astype(jnp.float32)
                wx = (1.0 - fx, fx)
                wy = (1.0 - fy, fy)
                wz = (1.0 - fz, fz)
                wxy = [wx[i & 1] * wy[(i >> 1) & 1] for i in range(4)]
                acc0 = None
                acc1 = None
                for corner in range(8):
                    w = wxy[corner & 3] * wz[(corner >> 2) & 1]
                    rows = (corner * CHUNK + s) + iota
                    f0 = plsc.load_gather(gat_ref, [rows, col0])
                    f1 = plsc.load_gather(gat_ref, [rows, col1])
                    if acc0 is None:
                        acc0, acc1 = w * f0, w * f1
                    else:
                        acc0 = acc0 + w * f0
                        acc1 = acc1 + w * f1
                prow = s + iota
                cc0 = jnp.full((16,), 2 * level, jnp.int32)
                cc1 = jnp.full((16,), 2 * level + 1, jnp.int32)
                plsc.store_scatter(out_v, [prow, cc0], acc0)
                plsc.store_scatter(out_v, [prow, cc1], acc1)
                return c

            lax.fori_loop(jnp.int32(0), jnp.int32(NGROUP), g_body, 0)

        gen_idx(0, idx_bufs[0])
        cps = [fire(0), None]
        for level in range(NUM_LEVELS):
            slot = level & 1
            if level + 1 < NUM_LEVELS:
                nxt = slot ^ 1
                gen_idx(level + 1, idx_bufs[nxt])
                cps[nxt] = fire(nxt)
            cps[slot].wait()
            accum(level, gat_bufs[slot])

        pltpu.sync_copy(out_v, out_hbm.at[pl.ds(base, CHUNK)])
        return carry

    lax.fori_loop(jnp.int32(0), jnp.int32(NCHUNKS), chunk_body, 0)


@jax.jit
def kernel(inputs, embeddings):
    mesh = plsc.VectorSubcoreMesh(core_axis_name="c", subcore_axis_name="s")
    k = functools.partial(
        pl.kernel,
        mesh=mesh,
        out_type=jax.ShapeDtypeStruct((BATCH, NUM_LEVELS * LEVEL_DIM),
                                      jnp.float32),
        compiler_params=pltpu.CompilerParams(needs_layout_passes=False,
                                             use_tc_tiling_on_sc=False),
        scratch_types=[
            pltpu.VMEM((CHUNK,), jnp.float32),
            pltpu.VMEM((CHUNK,), jnp.float32),
            pltpu.VMEM((CHUNK,), jnp.float32),
            pltpu.VMEM((NIDX,), jnp.int32),
            pltpu.VMEM((NIDX,), jnp.int32),
            pltpu.VMEM((NIDX, LEVEL_DIM), jnp.float32),
            pltpu.VMEM((NIDX, LEVEL_DIM), jnp.float32),
            pltpu.VMEM((CHUNK, NUM_LEVELS * LEVEL_DIM), jnp.float32),
            pltpu.SemaphoreType.DMA,
            pltpu.SemaphoreType.DMA,
        ],
    )(_sc_body)
    xs = inputs[:, 0]
    ys = inputs[:, 1]
    zs = inputs[:, 2]
    q = TOTAL_ROWS // 128
    c0, c1 = lax.optimization_barrier((embeddings[:, 0], embeddings[:, 1]))
    a = c0.reshape(q, 128)
    b = c1.reshape(q, 128)
    ai = lax.pad(a, jnp.float32(0), ((0, 0, 0), (0, 1, 1)))
    bi = lax.pad(b, jnp.float32(0), ((0, 0, 0), (1, 0, 1)))
    embi = (ai + bi).reshape(TOTAL_ROWS, LEVEL_DIM)
    return k(xs, ys, zs, embi)


# speedup vs baseline: 2.3415x; 2.3415x over previous
import functools

import numpy as np
import jax
import jax.numpy as jnp
from jax import lax
from jax.experimental import pallas as pl
from jax.experimental.pallas import tpu as pltpu
from jax.experimental.pallas import tpu_sc as plsc

INPUT_DIM = 3
NUM_LEVELS = 16
LEVEL_DIM = 2
BASE_RES = 16
LOG2_HASHMAP = 19
BATCH = 262144

_RES = [BASE_RES * 2 ** i for i in range(NUM_LEVELS)]
_PARAMS = []
_OFFSET = []
_off = 0
for _i in range(NUM_LEVELS):
    _p = min(2 ** LOG2_HASHMAP, _RES[_i] ** INPUT_DIM)
    _p = int(np.ceil(_p / 32) * 32)
    _PARAMS.append(_p)
    _OFFSET.append(_off)
    _off += _p
TOTAL_ROWS = _off

_P1 = int(np.uint32(2654435761).astype(np.int32))
_P2 = int(np.uint32(805459861).astype(np.int32))

NC, NS = 2, 16
NW = NC * NS
CHUNK = 512
PW = BATCH // NW
NCHUNKS = PW // CHUNK
NIDX = 8 * CHUNK
NGROUP = CHUNK // 16


def _sc_body(x_hbm, y_hbm, z_hbm, emb_hbm, out_hbm,
             xv, yv, zv, idx_a, idx_b, gat_a, gat_b, out_v, sem_a, sem_b):
    wid = lax.axis_index("s") * NC + lax.axis_index("c")
    iota = lax.iota(jnp.int32, 16)
    col0 = jnp.zeros((16,), jnp.int32)
    col1 = jnp.ones((16,), jnp.int32)
    idx_bufs = (idx_a, idx_b)
    gat_bufs = (gat_a, gat_b)
    sems = (sem_a, sem_b)

    def chunk_body(ci, carry):
        base = (wid * NCHUNKS + ci) * CHUNK
        pltpu.sync_copy(x_hbm.at[pl.ds(base, CHUNK)], xv)
        pltpu.sync_copy(y_hbm.at[pl.ds(base, CHUNK)], yv)
        pltpu.sync_copy(z_hbm.at[pl.ds(base, CHUNK)], zv)

        def gen_idx(level, idx_ref):
            res = float(_RES[level])
            mask = _PARAMS[level] - 1
            off = _OFFSET[level]

            def g_body(g, c):
                s = g * 16
                xi = (xv[pl.ds(s, 16)] * res).astype(jnp.int32)
                yi = (yv[pl.ds(s, 16)] * res).astype(jnp.int32)
                zi = (zv[pl.ds(s, 16)] * res).astype(jnp.int32)
                a0, b0 = xi, xi + 1
                a1 = yi * _P1
                b1 = a1 + _P1
                a2 = zi * _P2
                b2 = a2 + _P2
                e00 = a0 ^ a1
                e10 = b0 ^ a1
                e01 = a0 ^ b1
                e11 = b0 ^ b1
                pairs = (e00, e10, e01, e11)
                for corner in range(8):
                    h = pairs[corner & 3] ^ (b2 if corner & 4 else a2)
                    i0 = ((h & mask) + off) << 1
                    idx_ref[pl.ds(corner * CHUNK + s, 16)] = i0
                    idx_ref[pl.ds(NIDX + corner * CHUNK + s, 16)] = i0 + 1
                return c

            lax.fori_loop(jnp.int32(0), jnp.int32(NGROUP), g_body, 0)

        def fire(slot):
            return pltpu.async_copy(emb_hbm.at[idx_bufs[slot]],
                                    gat_bufs[slot], sems[slot])

        def accum(level, gat_ref):
            res = float(_RES[level])

            def g_body(g, c):
                s = g * 16
                x = xv[pl.ds(s, 16)] * res
                y = yv[pl.ds(s, 16)] * res
                z = zv[pl.ds(s, 16)] * res
                fx = x - x.astype(jnp.int32).astype(jnp.float32)
                fy = y - y.astype(jnp.int32).astype(jnp.float32)
                fz = z - z.astype(jnp.int32).astype(jnp.float32)
                wx = (1.0 - fx, fx)
                wy = (1.0 - fy, fy)
                wz = (1.0 - fz, fz)
                wxy = [wx[i & 1] * wy[(i >> 1) & 1] for i in range(4)]
                acc0 = None
                acc1 = None
                for corner in range(8):
                    w = wxy[corner & 3] * wz[(corner >> 2) & 1]
                    f0 = gat_ref[pl.ds(corner * CHUNK + s, 16)]
                    f1 = gat_ref[pl.ds(NIDX + corner * CHUNK + s, 16)]
                    if acc0 is None:
                        acc0, acc1 = w * f0, w * f1
                    else:
                        acc0 = acc0 + w * f0
                        acc1 = acc1 + w * f1
                prow = s + iota
                cc0 = jnp.full((16,), 2 * level, jnp.int32)
                cc1 = jnp.full((16,), 2 * level + 1, jnp.int32)
                plsc.store_scatter(out_v, [prow, cc0], acc0)
                plsc.store_scatter(out_v, [prow, cc1], acc1)
                return c

            lax.fori_loop(jnp.int32(0), jnp.int32(NGROUP), g_body, 0)

        gen_idx(0, idx_bufs[0])
        cps = [fire(0), None]
        for level in range(NUM_LEVELS):
            slot = level & 1
            if level + 1 < NUM_LEVELS:
                nxt = slot ^ 1
                gen_idx(level + 1, idx_bufs[nxt])
                cps[nxt] = fire(nxt)
            cps[slot].wait()
            accum(level, gat_bufs[slot])

        pltpu.sync_copy(out_v, out_hbm.at[pl.ds(base, CHUNK)])
        return carry

    lax.fori_loop(jnp.int32(0), jnp.int32(NCHUNKS), chunk_body, 0)


@jax.jit
def kernel(inputs, embeddings):
    mesh = plsc.VectorSubcoreMesh(core_axis_name="c", subcore_axis_name="s")
    k = functools.partial(
        pl.kernel,
        mesh=mesh,
        out_type=jax.ShapeDtypeStruct((BATCH, NUM_LEVELS * LEVEL_DIM),
                                      jnp.float32),
        compiler_params=pltpu.CompilerParams(needs_layout_passes=False,
                                             use_tc_tiling_on_sc=False),
        scratch_types=[
            pltpu.VMEM((CHUNK,), jnp.float32),
            pltpu.VMEM((CHUNK,), jnp.float32),
            pltpu.VMEM((CHUNK,), jnp.float32),
            pltpu.VMEM((2 * NIDX,), jnp.int32),
            pltpu.VMEM((2 * NIDX,), jnp.int32),
            pltpu.VMEM((2 * NIDX,), jnp.float32),
            pltpu.VMEM((2 * NIDX,), jnp.float32),
            pltpu.VMEM((CHUNK, NUM_LEVELS * LEVEL_DIM), jnp.float32),
            pltpu.SemaphoreType.DMA,
            pltpu.SemaphoreType.DMA,
        ],
    )(_sc_body)
    xs = inputs[:, 0]
    ys = inputs[:, 1]
    zs = inputs[:, 2]
    m = TOTAL_ROWS // 64
    c0, c1 = lax.optimization_barrier((embeddings[:, 0], embeddings[:, 1]))
    a = c0.reshape(m, 64)
    b = c1.reshape(m, 64)
    ai = lax.pad(a, jnp.float32(0), ((0, 0, 0), (0, 1, 1)))
    bi = lax.pad(b, jnp.float32(0), ((0, 0, 0), (1, 0, 1)))
    embi = (ai + bi).reshape(2 * TOTAL_ROWS)
    return k(xs, ys, zs, embi)


# speedup vs baseline: 3.9175x; 1.6731x over previous
import functools

import numpy as np
import jax
import jax.numpy as jnp
from jax import lax
from jax.experimental import pallas as pl
from jax.experimental.pallas import tpu as pltpu
from jax.experimental.pallas import tpu_sc as plsc

INPUT_DIM = 3
NUM_LEVELS = 16
LEVEL_DIM = 2
BASE_RES = 16
LOG2_HASHMAP = 19
BATCH = 262144

_RES = [BASE_RES * 2 ** i for i in range(NUM_LEVELS)]
_PARAMS = []
_OFFSET = []
_off = 0
for _i in range(NUM_LEVELS):
    _p = min(2 ** LOG2_HASHMAP, _RES[_i] ** INPUT_DIM)
    _p = int(np.ceil(_p / 32) * 32)
    _PARAMS.append(_p)
    _OFFSET.append(_off)
    _off += _p
TOTAL_ROWS = _off

_P1 = int(np.uint32(2654435761).astype(np.int32))
_P2 = int(np.uint32(805459861).astype(np.int32))

NC, NS = 2, 16
NW = NC * NS
CHUNK = 512
PW = BATCH // NW
NCHUNKS = PW // CHUNK
NIDX = 8 * CHUNK
NGROUP = CHUNK // 16


def _sc_body(x_hbm, y_hbm, z_hbm, emb_hbm, out_hbm,
             xv, yv, zv, idx_a, idx_b, gat_a, gat_b, out_v, sem_a, sem_b):
    wid = lax.axis_index("s") * NC + lax.axis_index("c")
    iota = lax.iota(jnp.int32, 16)
    col0 = jnp.zeros((16,), jnp.int32)
    col1 = jnp.ones((16,), jnp.int32)
    idx_bufs = (idx_a, idx_b)
    gat_bufs = (gat_a, gat_b)
    sems = (sem_a, sem_b)

    def chunk_body(ci, carry):
        base = (wid * NCHUNKS + ci) * CHUNK
        pltpu.sync_copy(x_hbm.at[pl.ds(base, CHUNK)], xv)
        pltpu.sync_copy(y_hbm.at[pl.ds(base, CHUNK)], yv)
        pltpu.sync_copy(z_hbm.at[pl.ds(base, CHUNK)], zv)

        def gen_idx(level, idx_ref):
            res = float(_RES[level])
            mask = _PARAMS[level] - 1
            off = _OFFSET[level]

            def g_body(g, c):
                s = g * 16
                xi = (xv[pl.ds(s, 16)] * res).astype(jnp.int32)
                yi = (yv[pl.ds(s, 16)] * res).astype(jnp.int32)
                zi = (zv[pl.ds(s, 16)] * res).astype(jnp.int32)
                a0, b0 = xi, xi + 1
                a1 = yi * _P1
                b1 = a1 + _P1
                a2 = zi * _P2
                b2 = a2 + _P2
                e00 = a0 ^ a1
                e10 = b0 ^ a1
                e01 = a0 ^ b1
                e11 = b0 ^ b1
                pairs = (e00, e10, e01, e11)
                for corner in range(8):
                    h = pairs[corner & 3] ^ (b2 if corner & 4 else a2)
                    i0 = ((h & mask) + off) << 1
                    idx_ref[pl.ds(corner * CHUNK + s, 16)] = i0
                    idx_ref[pl.ds(NIDX + corner * CHUNK + s, 16)] = i0 + 1
                return c

            lax.fori_loop(jnp.int32(0), jnp.int32(NGROUP), g_body, 0)

        def fire(slot):
            return pltpu.async_copy(emb_hbm.at[idx_bufs[slot]],
                                    gat_bufs[slot], sems[slot])

        def accum(level, gat_ref):
            res = float(_RES[level])

            def g_body(g, c):
                s = g * 16
                x = xv[pl.ds(s, 16)] * res
                y = yv[pl.ds(s, 16)] * res
                z = zv[pl.ds(s, 16)] * res
                fx = x - x.astype(jnp.int32).astype(jnp.float32)
                fy = y - y.astype(jnp.int32).astype(jnp.float32)
                fz = z - z.astype(jnp.int32).astype(jnp.float32)
                wx = (1.0 - fx, fx)
                wy = (1.0 - fy, fy)
                wz = (1.0 - fz, fz)
                wxy = [wx[i & 1] * wy[(i >> 1) & 1] for i in range(4)]
                acc0 = None
                acc1 = None
                for corner in range(8):
                    w = wxy[corner & 3] * wz[(corner >> 2) & 1]
                    f0 = gat_ref[pl.ds(corner * CHUNK + s, 16)]
                    f1 = gat_ref[pl.ds(NIDX + corner * CHUNK + s, 16)]
                    if acc0 is None:
                        acc0, acc1 = w * f0, w * f1
                    else:
                        acc0 = acc0 + w * f0
                        acc1 = acc1 + w * f1
                prow = s + iota
                cc0 = jnp.full((16,), 2 * level, jnp.int32)
                cc1 = jnp.full((16,), 2 * level + 1, jnp.int32)
                plsc.store_scatter(out_v, [prow, cc0], acc0)
                plsc.store_scatter(out_v, [prow, cc1], acc1)
                return c

            lax.fori_loop(jnp.int32(0), jnp.int32(NGROUP), g_body, 0)

        gen_idx(0, idx_bufs[0])
        cps = [fire(0), None]
        for level in range(NUM_LEVELS):
            slot = level & 1
            if level + 1 < NUM_LEVELS:
                nxt = slot ^ 1
                gen_idx(level + 1, idx_bufs[nxt])
                cps[nxt] = fire(nxt)
            cps[slot].wait()
            accum(level, gat_bufs[slot])

        pltpu.sync_copy(out_v, out_hbm.at[pl.ds(base, CHUNK)])
        return carry

    lax.fori_loop(jnp.int32(0), jnp.int32(NCHUNKS), chunk_body, 0)


@jax.jit
def kernel(inputs, embeddings):
    mesh = plsc.VectorSubcoreMesh(core_axis_name="c", subcore_axis_name="s")
    k = functools.partial(
        pl.kernel,
        mesh=mesh,
        out_type=jax.ShapeDtypeStruct((BATCH, NUM_LEVELS * LEVEL_DIM),
                                      jnp.float32),
        compiler_params=pltpu.CompilerParams(needs_layout_passes=False,
                                             use_tc_tiling_on_sc=False),
        scratch_types=[
            pltpu.VMEM((CHUNK,), jnp.float32),
            pltpu.VMEM((CHUNK,), jnp.float32),
            pltpu.VMEM((CHUNK,), jnp.float32),
            pltpu.VMEM((2 * NIDX,), jnp.int32),
            pltpu.VMEM((2 * NIDX,), jnp.int32),
            pltpu.VMEM((2 * NIDX,), jnp.float32),
            pltpu.VMEM((2 * NIDX,), jnp.float32),
            pltpu.VMEM((CHUNK, NUM_LEVELS * LEVEL_DIM), jnp.float32),
            pltpu.SemaphoreType.DMA,
            pltpu.SemaphoreType.DMA,
        ],
    )(_sc_body)
    xs = inputs[:, 0]
    ys = inputs[:, 1]
    zs = inputs[:, 2]
    m = TOTAL_ROWS // 64
    c0, c1 = lax.optimization_barrier((embeddings[:, 0], embeddings[:, 1]))
    a = c0.reshape(m, 64)
    b = c1.reshape(m, 64)
    w0 = np.zeros((64, 128), np.float32)
    w0[np.arange(64), 2 * np.arange(64)] = 1.0
    w1 = np.zeros((64, 128), np.float32)
    w1[np.arange(64), 2 * np.arange(64) + 1] = 1.0
    embi = (
        jnp.dot(a, jnp.asarray(w0), precision=lax.Precision.HIGHEST)
        + jnp.dot(b, jnp.asarray(w1), precision=lax.Precision.HIGHEST)
    ).reshape(2 * TOTAL_ROWS)
    return k(xs, ys, zs, embi)


# speedup vs baseline: 5.1494x; 1.3144x over previous
import functools

import numpy as np
import jax
import jax.numpy as jnp
from jax import lax
from jax.experimental import pallas as pl
from jax.experimental.pallas import tpu as pltpu
from jax.experimental.pallas import tpu_sc as plsc

INPUT_DIM = 3
NUM_LEVELS = 16
LEVEL_DIM = 2
BASE_RES = 16
LOG2_HASHMAP = 19
BATCH = 262144

_RES = [BASE_RES * 2 ** i for i in range(NUM_LEVELS)]
_PARAMS = []
_OFFSET = []
_off = 0
for _i in range(NUM_LEVELS):
    _p = min(2 ** LOG2_HASHMAP, _RES[_i] ** INPUT_DIM)
    _p = int(np.ceil(_p / 32) * 32)
    _PARAMS.append(_p)
    _OFFSET.append(_off)
    _off += _p
TOTAL_ROWS = _off

_P1 = int(np.uint32(2654435761).astype(np.int32))
_P2 = int(np.uint32(805459861).astype(np.int32))

NC, NS = 2, 16
NW = NC * NS
CHUNK = 512
PW = BATCH // NW
NCHUNKS = PW // CHUNK
NIDX = 8 * CHUNK
NGROUP = CHUNK // 16


def _sc_body(x_hbm, y_hbm, z_hbm, emb_hbm, out_hbm,
             xv, yv, zv, idx_a, idx_b, gat_a, gat_b, out_v, sem_a, sem_b):
    wid = lax.axis_index("s") * NC + lax.axis_index("c")
    iota = lax.iota(jnp.int32, 16)
    col0 = jnp.zeros((16,), jnp.int32)
    col1 = jnp.ones((16,), jnp.int32)
    idx_bufs = (idx_a, idx_b)
    gat_bufs = (gat_a, gat_b)
    sems = (sem_a, sem_b)

    def chunk_body(ci, carry):
        base = (wid * NCHUNKS + ci) * CHUNK
        pltpu.sync_copy(x_hbm.at[pl.ds(base, CHUNK)], xv)
        pltpu.sync_copy(y_hbm.at[pl.ds(base, CHUNK)], yv)
        pltpu.sync_copy(z_hbm.at[pl.ds(base, CHUNK)], zv)

        def gen_idx(level, idx_ref):
            res = float(_RES[level])
            mask = _PARAMS[level] - 1
            off = _OFFSET[level]

            def g_body(g, c):
                s = g * 16
                xi = (xv[pl.ds(s, 16)] * res).astype(jnp.int32)
                yi = (yv[pl.ds(s, 16)] * res).astype(jnp.int32)
                zi = (zv[pl.ds(s, 16)] * res).astype(jnp.int32)
                a0, b0 = xi, xi + 1
                a1 = yi * _P1
                b1 = a1 + _P1
                a2 = zi * _P2
                b2 = a2 + _P2
                e00 = a0 ^ a1
                e10 = b0 ^ a1
                e01 = a0 ^ b1
                e11 = b0 ^ b1
                pairs = (e00, e10, e01, e11)
                for corner in range(8):
                    h = pairs[corner & 3] ^ (b2 if corner & 4 else a2)
                    idx_ref[pl.ds(corner * CHUNK + s, 16)] = (h & mask) + off
                return c

            lax.fori_loop(jnp.int32(0), jnp.int32(NGROUP), g_body, 0)

        def fire(slot):
            return pltpu.async_copy(emb_hbm.at[idx_bufs[slot]],
                                    gat_bufs[slot], sems[slot])

        def accum(level, gat_ref):
            res = float(_RES[level])

            def g_body(g, c):
                s = g * 16
                x = xv[pl.ds(s, 16)] * res
                y = yv[pl.ds(s, 16)] * res
                z = zv[pl.ds(s, 16)] * res
                fx = x - x.astype(jnp.int32).astype(jnp.float32)
                fy = y - y.astype(jnp.int32).astype(jnp.float32)
                fz = z - z.astype(jnp.int32).astype(jnp.float32)
                wx = (1.0 - fx, fx)
                wy = (1.0 - fy, fy)
                wz = (1.0 - fz, fz)
                wxy = [wx[i & 1] * wy[(i >> 1) & 1] for i in range(4)]
                acc0 = None
                acc1 = None
                for corner in range(8):
                    w = wxy[corner & 3] * wz[(corner >> 2) & 1]
                    rows = (corner * CHUNK + s) + iota
                    f0 = plsc.load_gather(gat_ref, [rows, col0])
                    f1 = plsc.load_gather(gat_ref, [rows, col1])
                    if acc0 is None:
                        acc0, acc1 = w * f0, w * f1
                    else:
                        acc0 = acc0 + w * f0
                        acc1 = acc1 + w * f1
                prow = s + iota
                cc0 = jnp.full((16,), 2 * level, jnp.int32)
                cc1 = jnp.full((16,), 2 * level + 1, jnp.int32)
                plsc.store_scatter(out_v, [prow, cc0], acc0)
                plsc.store_scatter(out_v, [prow, cc1], acc1)
                return c

            lax.fori_loop(jnp.int32(0), jnp.int32(NGROUP), g_body, 0)

        gen_idx(0, idx_bufs[0])
        cps = [fire(0), None]
        for level in range(NUM_LEVELS):
            slot = level & 1
            if level + 1 < NUM_LEVELS:
                nxt = slot ^ 1
                gen_idx(level + 1, idx_bufs[nxt])
                cps[nxt] = fire(nxt)
            cps[slot].wait()
            accum(level, gat_bufs[slot])

        pltpu.sync_copy(out_v, out_hbm.at[pl.ds(base, CHUNK)])
        return carry

    lax.fori_loop(jnp.int32(0), jnp.int32(NCHUNKS), chunk_body, 0)


@jax.jit
def kernel(inputs, embeddings):
    mesh = plsc.VectorSubcoreMesh(core_axis_name="c", subcore_axis_name="s")
    k = functools.partial(
        pl.kernel,
        mesh=mesh,
        out_type=jax.ShapeDtypeStruct((BATCH, NUM_LEVELS * LEVEL_DIM),
                                      jnp.float32),
        compiler_params=pltpu.CompilerParams(needs_layout_passes=False,
                                             use_tc_tiling_on_sc=False),
        scratch_types=[
            pltpu.VMEM((CHUNK,), jnp.float32),
            pltpu.VMEM((CHUNK,), jnp.float32),
            pltpu.VMEM((CHUNK,), jnp.float32),
            pltpu.VMEM((NIDX,), jnp.int32),
            pltpu.VMEM((NIDX,), jnp.int32),
            pltpu.VMEM((NIDX, 8), jnp.float32),
            pltpu.VMEM((NIDX, 8), jnp.float32),
            pltpu.VMEM((CHUNK, NUM_LEVELS * LEVEL_DIM), jnp.float32),
            pltpu.SemaphoreType.DMA,
            pltpu.SemaphoreType.DMA,
        ],
    )(_sc_body)
    xs = inputs[:, 0]
    ys = inputs[:, 1]
    zs = inputs[:, 2]
    m = TOTAL_ROWS // 16
    c0, c1 = lax.optimization_barrier((embeddings[:, 0], embeddings[:, 1]))
    a = c0.reshape(m, 16)
    b = c1.reshape(m, 16)
    w0 = np.zeros((16, 128), np.float32)
    w0[np.arange(16), 8 * np.arange(16)] = 1.0
    w1 = np.zeros((16, 128), np.float32)
    w1[np.arange(16), 8 * np.arange(16) + 1] = 1.0
    embi = (
        jnp.dot(a, jnp.asarray(w0), precision=lax.Precision.HIGHEST)
        + jnp.dot(b, jnp.asarray(w1), precision=lax.Precision.HIGHEST)
    ).reshape(TOTAL_ROWS, 8)
    return k(xs, ys, zs, embi)


# speedup vs baseline: 5.7714x; 1.1208x over previous
import functools

import numpy as np
import jax
import jax.numpy as jnp
from jax import lax
from jax.experimental import pallas as pl
from jax.experimental.pallas import tpu as pltpu
from jax.experimental.pallas import tpu_sc as plsc

INPUT_DIM = 3
NUM_LEVELS = 16
LEVEL_DIM = 2
BASE_RES = 16
LOG2_HASHMAP = 19
BATCH = 262144

_RES = [BASE_RES * 2 ** i for i in range(NUM_LEVELS)]
_PARAMS = []
_OFFSET = []
_off = 0
for _i in range(NUM_LEVELS):
    _p = min(2 ** LOG2_HASHMAP, _RES[_i] ** INPUT_DIM)
    _p = int(np.ceil(_p / 32) * 32)
    _PARAMS.append(_p)
    _OFFSET.append(_off)
    _off += _p
TOTAL_ROWS = _off

_P1 = int(np.uint32(2654435761).astype(np.int32))
_P2 = int(np.uint32(805459861).astype(np.int32))

NC, NS = 2, 16
NW = NC * NS
CHUNK = 256
PW = BATCH // NW
NCHUNKS = PW // CHUNK
NIDX = 8 * CHUNK
NGROUP = CHUNK // 16
NROWS = TOTAL_ROWS // 4
N_LOCAL = 2
_T0 = _PARAMS[0] // 4
_T1 = _PARAMS[1] // 4


def _sc_body(x_hbm, y_hbm, z_hbm, emb_hbm, out_hbm,
             xv, yv, zv, tab0, tab1, idx_a, idx_b, pp_a, pp_b,
             gat_a, gat_b, out_v, sem_a, sem_b):
    wid = lax.axis_index("s") * NC + lax.axis_index("c")
    iota = lax.iota(jnp.int32, 16)
    idx_bufs = (idx_a, idx_b)
    pp_bufs = (pp_a, pp_b)
    gat_bufs = (gat_a, gat_b)
    sems = (sem_a, sem_b)

    pltpu.sync_copy(emb_hbm.at[pl.ds(0, _T0)], tab0)
    pltpu.sync_copy(emb_hbm.at[pl.ds(_T0, _T1)], tab1)

    def corner_hashes(s, res):
        xi = (xv[pl.ds(s, 16)] * res).astype(jnp.int32)
        yi = (yv[pl.ds(s, 16)] * res).astype(jnp.int32)
        zi = (zv[pl.ds(s, 16)] * res).astype(jnp.int32)
        a1 = yi * _P1
        b1 = a1 + _P1
        a2 = zi * _P2
        b2 = a2 + _P2
        e00 = xi ^ a1
        e10 = (xi + 1) ^ a1
        e01 = xi ^ b1
        e11 = (xi + 1) ^ b1
        pairs = (e00, e10, e01, e11)
        return [pairs[c & 3] ^ (b2 if c & 4 else a2) for c in range(8)]

    def weights8(s, res):
        x = xv[pl.ds(s, 16)] * res
        y = yv[pl.ds(s, 16)] * res
        z = zv[pl.ds(s, 16)] * res
        fx = x - x.astype(jnp.int32).astype(jnp.float32)
        fy = y - y.astype(jnp.int32).astype(jnp.float32)
        fz = z - z.astype(jnp.int32).astype(jnp.float32)
        wx = (1.0 - fx, fx)
        wy = (1.0 - fy, fy)
        wz = (1.0 - fz, fz)
        wxy = [wx[i & 1] * wy[(i >> 1) & 1] for i in range(4)]
        return [wxy[c & 3] * wz[(c >> 2) & 1] for c in range(8)]

    def out_store(s, level, acc0, acc1):
        prow = s + iota
        cc0 = jnp.full((16,), 2 * level, jnp.int32)
        cc1 = jnp.full((16,), 2 * level + 1, jnp.int32)
        plsc.store_scatter(out_v, [prow, cc0], acc0)
        plsc.store_scatter(out_v, [prow, cc1], acc1)

    def chunk_body(ci, carry):
        base = (wid * NCHUNKS + ci) * CHUNK
        pltpu.sync_copy(x_hbm.at[pl.ds(base, CHUNK)], xv)
        pltpu.sync_copy(y_hbm.at[pl.ds(base, CHUNK)], yv)
        pltpu.sync_copy(z_hbm.at[pl.ds(base, CHUNK)], zv)

        def gen_idx(level, idx_ref, pp_ref):
            res = float(_RES[level])
            mask = _PARAMS[level] - 1
            off4 = _OFFSET[level] // 4

            def g_body(g, c):
                s = g * 16
                hs = corner_hashes(s, res)
                for corner in range(8):
                    t = hs[corner] & mask
                    idx_ref[pl.ds(corner * CHUNK + s, 16)] = (t >> 2) + off4
                    pp_ref[pl.ds(corner * CHUNK + s, 16)] = (t & 3) << 1
                return c

            lax.fori_loop(jnp.int32(0), jnp.int32(NGROUP), g_body, 0)

        def fire(slot):
            return pltpu.async_copy(emb_hbm.at[idx_bufs[slot]],
                                    gat_bufs[slot], sems[slot])

        def accum_stream(level, gat_ref, pp_ref):
            res = float(_RES[level])

            def g_body(g, c):
                s = g * 16
                ws = weights8(s, res)
                acc0 = None
                acc1 = None
                for corner in range(8):
                    rows = (corner * CHUNK + s) + iota
                    cl0 = pp_ref[pl.ds(corner * CHUNK + s, 16)]
                    f0 = plsc.load_gather(gat_ref, [rows, cl0])
                    f1 = plsc.load_gather(gat_ref, [rows, cl0 + 1])
                    w = ws[corner]
                    if acc0 is None:
                        acc0, acc1 = w * f0, w * f1
                    else:
                        acc0 = acc0 + w * f0
                        acc1 = acc1 + w * f1
                out_store(s, level, acc0, acc1)
                return c

            lax.fori_loop(jnp.int32(0), jnp.int32(NGROUP), g_body, 0)

        def accum_local(level, tab_ref):
            res = float(_RES[level])
            mask = _PARAMS[level] - 1

            def g_body(g, c):
                s = g * 16
                hs = corner_hashes(s, res)
                ws = weights8(s, res)
                acc0 = None
                acc1 = None
                for corner in range(8):
                    t = hs[corner] & mask
                    rows = t >> 2
                    cl0 = (t & 3) << 1
                    f0 = plsc.load_gather(tab_ref, [rows, cl0])
                    f1 = plsc.load_gather(tab_ref, [rows, cl0 + 1])
                    w = ws[corner]
                    if acc0 is None:
                        acc0, acc1 = w * f0, w * f1
                    else:
                        acc0 = acc0 + w * f0
                        acc1 = acc1 + w * f1
                out_store(s, level, acc0, acc1)
                return c

            lax.fori_loop(jnp.int32(0), jnp.int32(NGROUP), g_body, 0)

        gen_idx(N_LOCAL, idx_bufs[0], pp_bufs[0])
        cps = [fire(0), None]
        accum_local(0, tab0)
        accum_local(1, tab1)
        for level in range(N_LOCAL, NUM_LEVELS):
            slot = level & 1
            if level + 1 < NUM_LEVELS:
                nxt = slot ^ 1
                gen_idx(level + 1, idx_bufs[nxt], pp_bufs[nxt])
                cps[nxt] = fire(nxt)
            cps[slot].wait()
            accum_stream(level, gat_bufs[slot], pp_bufs[slot])

        pltpu.sync_copy(out_v, out_hbm.at[pl.ds(base, CHUNK)])
        return carry

    lax.fori_loop(jnp.int32(0), jnp.int32(NCHUNKS), chunk_body, 0)


@jax.jit
def kernel(inputs, embeddings):
    mesh = plsc.VectorSubcoreMesh(core_axis_name="c", subcore_axis_name="s")
    k = functools.partial(
        pl.kernel,
        mesh=mesh,
        out_type=jax.ShapeDtypeStruct((BATCH, NUM_LEVELS * LEVEL_DIM),
                                      jnp.float32),
        compiler_params=pltpu.CompilerParams(needs_layout_passes=False,
                                             use_tc_tiling_on_sc=False),
        scratch_types=[
            pltpu.VMEM((CHUNK,), jnp.float32),
            pltpu.VMEM((CHUNK,), jnp.float32),
            pltpu.VMEM((CHUNK,), jnp.float32),
            pltpu.VMEM((_T0, 8), jnp.float32),
            pltpu.VMEM((_T1, 8), jnp.float32),
            pltpu.VMEM((NIDX,), jnp.int32),
            pltpu.VMEM((NIDX,), jnp.int32),
            pltpu.VMEM((NIDX,), jnp.int32),
            pltpu.VMEM((NIDX,), jnp.int32),
            pltpu.VMEM((NIDX, 8), jnp.float32),
            pltpu.VMEM((NIDX, 8), jnp.float32),
            pltpu.VMEM((CHUNK, NUM_LEVELS * LEVEL_DIM), jnp.float32),
            pltpu.SemaphoreType.DMA,
            pltpu.SemaphoreType.DMA,
        ],
    )(_sc_body)
    xs = inputs[:, 0]
    ys = inputs[:, 1]
    zs = inputs[:, 2]
    m = TOTAL_ROWS // 64
    c0, c1 = lax.optimization_barrier(
        (embeddings[:, 0].reshape(m, 64), embeddings[:, 1].reshape(m, 64)))
    w0 = np.zeros((64, 128), np.float32)
    w0[np.arange(64), 2 * np.arange(64)] = 1.0
    w1 = np.zeros((64, 128), np.float32)
    w1[np.arange(64), 2 * np.arange(64) + 1] = 1.0
    embi = (
        jnp.dot(c0, jnp.asarray(w0), precision=lax.Precision.HIGHEST)
        + jnp.dot(c1, jnp.asarray(w1), precision=lax.Precision.HIGHEST)
    ).reshape(NROWS, 8)
    return k(xs, ys, zs, embi)


# speedup vs baseline: 5.7783x; 1.0012x over previous
import functools

import numpy as np
import jax
import jax.numpy as jnp
from jax import lax
from jax.experimental import pallas as pl
from jax.experimental.pallas import tpu as pltpu
from jax.experimental.pallas import tpu_sc as plsc

INPUT_DIM = 3
NUM_LEVELS = 16
LEVEL_DIM = 2
BASE_RES = 16
LOG2_HASHMAP = 19
BATCH = 262144

_RES = [BASE_RES * 2 ** i for i in range(NUM_LEVELS)]
_PARAMS = []
_OFFSET = []
_off = 0
for _i in range(NUM_LEVELS):
    _p = min(2 ** LOG2_HASHMAP, _RES[_i] ** INPUT_DIM)
    _p = int(np.ceil(_p / 32) * 32)
    _PARAMS.append(_p)
    _OFFSET.append(_off)
    _off += _p
TOTAL_ROWS = _off

_P1 = int(np.uint32(2654435761).astype(np.int32))
_P2 = int(np.uint32(805459861).astype(np.int32))

NC, NS = 2, 16
NW = NC * NS
CHUNK = 256
PW = BATCH // NW
NCHUNKS = PW // CHUNK
NIDX = 8 * CHUNK
NGROUP = CHUNK // 16
NROWS = TOTAL_ROWS // 4
N_LOCAL = 2
_T0 = _PARAMS[0] // 4
_T1 = _PARAMS[1] // 4


def _sc_body(x_hbm, y_hbm, z_hbm, emb_hbm, out_hbm,
             xv, yv, zv, tab0, tab1, idx_a, idx_b, pp_a, pp_b,
             gat_a, gat_b, out_v, sem_a, sem_b):
    wid = lax.axis_index("s") * NC + lax.axis_index("c")
    iota = lax.iota(jnp.int32, 16)
    idx_bufs = (idx_a, idx_b)
    pp_bufs = (pp_a, pp_b)
    gat_bufs = (gat_a, gat_b)
    sems = (sem_a, sem_b)

    pltpu.sync_copy(emb_hbm.at[pl.ds(0, _T0)], tab0)
    pltpu.sync_copy(emb_hbm.at[pl.ds(_T0, _T1)], tab1)

    def corner_hashes(s, res):
        xi = (xv[pl.ds(s, 16)] * res).astype(jnp.int32)
        yi = (yv[pl.ds(s, 16)] * res).astype(jnp.int32)
        zi = (zv[pl.ds(s, 16)] * res).astype(jnp.int32)
        a1 = yi * _P1
        b1 = a1 + _P1
        a2 = zi * _P2
        b2 = a2 + _P2
        e00 = xi ^ a1
        e10 = (xi + 1) ^ a1
        e01 = xi ^ b1
        e11 = (xi + 1) ^ b1
        pairs = (e00, e10, e01, e11)
        return [pairs[c & 3] ^ (b2 if c & 4 else a2) for c in range(8)]

    def weights8(s, res):
        x = xv[pl.ds(s, 16)] * res
        y = yv[pl.ds(s, 16)] * res
        z = zv[pl.ds(s, 16)] * res
        fx = x - x.astype(jnp.int32).astype(jnp.float32)
        fy = y - y.astype(jnp.int32).astype(jnp.float32)
        fz = z - z.astype(jnp.int32).astype(jnp.float32)
        wx = (1.0 - fx, fx)
        wy = (1.0 - fy, fy)
        wz = (1.0 - fz, fz)
        wxy = [wx[i & 1] * wy[(i >> 1) & 1] for i in range(4)]
        return [wxy[c & 3] * wz[(c >> 2) & 1] for c in range(8)]

    def out_store(s, level, acc0, acc1):
        prow = s + iota
        cc0 = jnp.full((16,), 2 * level, jnp.int32)
        cc1 = jnp.full((16,), 2 * level + 1, jnp.int32)
        plsc.store_scatter(out_v, [prow, cc0], acc0)
        plsc.store_scatter(out_v, [prow, cc1], acc1)

    def chunk_body(ci, carry):
        base = (wid * NCHUNKS + ci) * CHUNK
        pltpu.sync_copy(x_hbm.at[pl.ds(base, CHUNK)], xv)
        pltpu.sync_copy(y_hbm.at[pl.ds(base, CHUNK)], yv)
        pltpu.sync_copy(z_hbm.at[pl.ds(base, CHUNK)], zv)

        def gen_idx(level, idx_ref, pp_ref):
            res = float(_RES[level])
            mask = _PARAMS[level] - 1
            off4 = _OFFSET[level] // 4

            def g_body(g, c):
                s = g * 16
                hs = corner_hashes(s, res)
                for corner in range(8):
                    t = hs[corner] & mask
                    idx_ref[pl.ds(corner * CHUNK + s, 16)] = (t >> 2) + off4
                    pp_ref[pl.ds(corner * CHUNK + s, 16)] = (t & 3) << 1
                return c

            lax.fori_loop(jnp.int32(0), jnp.int32(NGROUP), g_body, 0)

        def fire(slot):
            return pltpu.async_copy(emb_hbm.at[idx_bufs[slot]],
                                    gat_bufs[slot], sems[slot])

        def accum_stream(level, gat_ref, pp_ref):
            res = float(_RES[level])

            def g_body(g, c):
                s = g * 16
                ws = weights8(s, res)
                acc0 = None
                acc1 = None
                for corner in range(8):
                    rows = (corner * CHUNK + s) + iota
                    cl0 = pp_ref[pl.ds(corner * CHUNK + s, 16)]
                    f0 = plsc.load_gather(gat_ref, [rows, cl0])
                    f1 = plsc.load_gather(gat_ref, [rows, cl0 + 1])
                    w = ws[corner]
                    if acc0 is None:
                        acc0, acc1 = w * f0, w * f1
                    else:
                        acc0 = acc0 + w * f0
                        acc1 = acc1 + w * f1
                out_store(s, level, acc0, acc1)
                return c

            lax.fori_loop(jnp.int32(0), jnp.int32(NGROUP), g_body, 0)

        def accum_local(level, tab_ref):
            res = float(_RES[level])
            mask = _PARAMS[level] - 1

            def g_body(g, c):
                s = g * 16
                hs = corner_hashes(s, res)
                ws = weights8(s, res)
                acc0 = None
                acc1 = None
                for corner in range(8):
                    t = hs[corner] & mask
                    rows = t >> 2
                    cl0 = (t & 3) << 1
                    f0 = plsc.load_gather(tab_ref, [rows, cl0])
                    f1 = plsc.load_gather(tab_ref, [rows, cl0 + 1])
                    w = ws[corner]
                    if acc0 is None:
                        acc0, acc1 = w * f0, w * f1
                    else:
                        acc0 = acc0 + w * f0
                        acc1 = acc1 + w * f1
                out_store(s, level, acc0, acc1)
                return c

            lax.fori_loop(jnp.int32(0), jnp.int32(NGROUP), g_body, 0)

        gen_idx(N_LOCAL, idx_bufs[0], pp_bufs[0])
        cps = [fire(0), None]
        accum_local(0, tab0)
        accum_local(1, tab1)
        for level in range(N_LOCAL, NUM_LEVELS):
            slot = level & 1
            if level + 1 < NUM_LEVELS:
                nxt = slot ^ 1
                gen_idx(level + 1, idx_bufs[nxt], pp_bufs[nxt])
                cps[nxt] = fire(nxt)
            cps[slot].wait()
            accum_stream(level, gat_bufs[slot], pp_bufs[slot])

        pltpu.sync_copy(out_v, out_hbm.at[pl.ds(base, CHUNK)])
        return carry

    lax.fori_loop(jnp.int32(0), jnp.int32(NCHUNKS), chunk_body, 0)


@jax.jit
def kernel(inputs, embeddings):
    mesh = plsc.VectorSubcoreMesh(core_axis_name="c", subcore_axis_name="s")
    k = functools.partial(
        pl.kernel,
        mesh=mesh,
        out_type=jax.ShapeDtypeStruct((BATCH, NUM_LEVELS * LEVEL_DIM),
                                      jnp.float32),
        compiler_params=pltpu.CompilerParams(needs_layout_passes=False,
                                             use_tc_tiling_on_sc=False),
        scratch_types=[
            pltpu.VMEM((CHUNK,), jnp.float32),
            pltpu.VMEM((CHUNK,), jnp.float32),
            pltpu.VMEM((CHUNK,), jnp.float32),
            pltpu.VMEM((_T0, 8), jnp.float32),
            pltpu.VMEM((_T1, 8), jnp.float32),
            pltpu.VMEM((NIDX,), jnp.int32),
            pltpu.VMEM((NIDX,), jnp.int32),
            pltpu.VMEM((NIDX,), jnp.int32),
            pltpu.VMEM((NIDX,), jnp.int32),
            pltpu.VMEM((NIDX, 8), jnp.float32),
            pltpu.VMEM((NIDX, 8), jnp.float32),
            pltpu.VMEM((CHUNK, NUM_LEVELS * LEVEL_DIM), jnp.float32),
            pltpu.SemaphoreType.DMA,
            pltpu.SemaphoreType.DMA,
        ],
    )(_sc_body)
    xs = inputs[:, 0]
    ys = inputs[:, 1]
    zs = inputs[:, 2]
    m = TOTAL_ROWS // 64
    c0 = embeddings[:, 0].reshape(m, 64)
    c1 = embeddings[:, 1].reshape(m, 64)
    w0 = np.zeros((64, 128), np.float32)
    w0[np.arange(64), 2 * np.arange(64)] = 1.0
    w1 = np.zeros((64, 128), np.float32)
    w1[np.arange(64), 2 * np.arange(64) + 1] = 1.0
    embi = (
        jnp.dot(c0, jnp.asarray(w0), precision=lax.Precision.HIGHEST)
        + jnp.dot(c1, jnp.asarray(w1), precision=lax.Precision.HIGHEST)
    ).reshape(NROWS, 8)
    return k(xs, ys, zs, embi)
